# baseline (speedup 1.0000x reference)
import functools

import jax
import jax.numpy as jnp
from jax import lax
from jax.experimental import pallas as pl
from jax.experimental.pallas import tpu as pltpu
from jax.experimental.pallas import tpu_sc as plsc

VOC = 100000
D = 128
B = 4096
CTX = 20


NC = 2
NS = 16
NW = NC * NS
ROWS_W = B // NW
CHUNK_R = 4
CHUNK_I = CHUNK_R * CTX
NCHUNK = ROWS_W // CHUNK_R
LANES = 16


def _sc_body(xf3, tab, out, idx_v, g, acc, sem):
    wid = lax.axis_index("s") * NC + lax.axis_index("c")
    pltpu.sync_copy(xf3.at[wid], idx_v)

    def chunk(k, carry):
        pltpu.async_copy(tab.at[idx_v.at[k]], g, sem).wait()
        for r in range(CHUNK_R):
            row = k * CHUNK_R + r
            for dc in range(D // LANES):
                v = g[r * CTX, pl.ds(dc * LANES, LANES)]
                for t in range(1, CTX):
                    v = v + g[r * CTX + t, pl.ds(dc * LANES, LANES)]
                acc[pl.ds(row * D + dc * LANES, LANES)] = v
        return carry

    lax.fori_loop(0, NCHUNK, chunk, 0)
    pltpu.sync_copy(acc, out.at[pl.ds(wid * ROWS_W * D, ROWS_W * D)])


@functools.cache
def _sc_gather_sum():
    return pl.kernel(
        _sc_body,
        out_type=jax.ShapeDtypeStruct((B * D,), jnp.float32),
        mesh=plsc.VectorSubcoreMesh(
            core_axis_name="c", subcore_axis_name="s",
            num_cores=NC, num_subcores=NS),
        scratch_types=[
            pltpu.VMEM((NCHUNK, CHUNK_I), jnp.int32),
            pltpu.VMEM((CHUNK_I, D), jnp.float32),
            pltpu.VMEM((ROWS_W * D,), jnp.float32),
            pltpu.SemaphoreType.DMA,
        ],
    )


A_BT = 2048
V_TILE = 2048
NV = -(-VOC // V_TILE)
V_PAD = NV * V_TILE
A_NB = B // A_BT


def _lse_body(esum_ref, w_ref, eb_ref, lse_ref, m_ref, s_ref):
    j = pl.program_id(1)
    logits = lax.dot_general(
        esum_ref[...], w_ref[...],
        (((1,), (1,)), ((), ())),
        preferred_element_type=jnp.float32,
    )

    @pl.when(j == 0)
    def _():
        m_ref[...] = jnp.full((A_BT, 1), -jnp.inf, jnp.float32)
        s_ref[...] = jnp.zeros((A_BT, 1), jnp.float32)

    tile_max = jnp.max(logits, axis=1, keepdims=True)
    new_m = jnp.maximum(m_ref[...], tile_max)
    t = jnp.exp(logits - new_m)
    part = lax.dot_general(
        t, eb_ref[...], (((1,), (1,)), ((), ())),
        preferred_element_type=jnp.float32)
    s_ref[...] = s_ref[...] * jnp.exp(m_ref[...] - new_m) + part
    m_ref[...] = new_m
    lse_ref[...] = m_ref[...] + jnp.log(s_ref[...])


BT = 1024
NB = B // BT
NSTEPS = NB * NV
V_LAST = VOC - (NV - 1) * V_TILE
V_LAST_A = V_LAST - (V_LAST % 128)
REM_BLK = VOC // 128


def _full_dma(buf, par, out_ref, i, j, sem):
    return pltpu.make_async_copy(
        buf.at[par], out_ref.at[pl.ds(i * BT, BT), pl.ds(j * V_TILE, V_TILE)],
        sem)


def _last_dma_a(buf, par, out_ref, i, sem):
    return pltpu.make_async_copy(
        buf.at[par, :, pl.ds(0, V_LAST_A)],
        out_ref.at[pl.ds(i * BT, BT),
                   pl.ds((NV - 1) * V_TILE, V_LAST_A)],
        sem)


def _wait_step(buf, out_ref, step, sems):
    par = lax.rem(step, 2)
    i2 = lax.div(step, NV)
    j2 = lax.rem(step, NV)

    @pl.when(jnp.logical_and(j2 < NV - 1, par == 0))
    def _():
        _full_dma(buf, 0, out_ref, i2, j2, sems[0]).wait()

    @pl.when(jnp.logical_and(j2 < NV - 1, par == 1))
    def _():
        _full_dma(buf, 1, out_ref, i2, j2, sems[1]).wait()

    @pl.when(jnp.logical_and(j2 == NV - 1, par == 0))
    def _():
        _last_dma_a(buf, 0, out_ref, i2, sems[0]).wait()

    @pl.when(jnp.logical_and(j2 == NV - 1, par == 1))
    def _():
        _last_dma_a(buf, 1, out_ref, i2, sems[1]).wait()


def _out_body(esum_ref, w_ref, b_ref, lse_ref, out_ref, buf, sem0, sem1):
    i = pl.program_id(0)
    j = pl.program_id(1)
    step = i * NV + j
    par = lax.rem(step, 2)
    sems = (sem0, sem1)

    @pl.when(step >= 2)
    def _():
        _wait_step(buf, out_ref, step - 2, sems)

    logits = lax.dot_general(
        esum_ref[...], w_ref[...],
        (((1,), (1,)), ((), ())),
        preferred_element_type=jnp.float32,
    )
    vals = (logits - lse_ref[...]) + b_ref[...]

    @pl.when(par == 0)
    def _():
        buf[0] = vals

    @pl.when(par == 1)
    def _():
        buf[1] = vals

    @pl.when(jnp.logical_and(j < NV - 1, par == 0))
    def _():
        _full_dma(buf, 0, out_ref, i, j, sem0).start()

    @pl.when(jnp.logical_and(j < NV - 1, par == 1))
    def _():
        _full_dma(buf, 1, out_ref, i, j, sem1).start()

    @pl.when(jnp.logical_and(j == NV - 1, par == 0))
    def _():
        _last_dma_a(buf, 0, out_ref, i, sem0).start()

    @pl.when(jnp.logical_and(j == NV - 1, par == 1))
    def _():
        _last_dma_a(buf, 1, out_ref, i, sem1).start()

    @pl.when(step == NSTEPS - 1)
    def _():
        _wait_step(buf, out_ref, step - 1, sems)
        _wait_step(buf, out_ref, step, sems)


def _rem_body(prev_ref, esum_ref, w_ref, b_ref, lse_ref, out_ref):
    del prev_ref
    logits = lax.dot_general(
        esum_ref[...], w_ref[...],
        (((1,), (1,)), ((), ())),
        preferred_element_type=jnp.float32,
    )
    out_ref[...] = (logits - lse_ref[...]) + b_ref[...]


def _tc_call(esum, w_bf, b2d, eb2d, interpret=False):
    lse = pl.pallas_call(
        _lse_body,
        grid=(A_NB, NV),
        in_specs=[
            pl.BlockSpec((A_BT, D), lambda i, j: (i, 0)),
            pl.BlockSpec((V_TILE, D), lambda i, j: (j, 0)),
            pl.BlockSpec((1, V_TILE), lambda i, j: (0, j)),
        ],
        out_specs=pl.BlockSpec((A_BT, 1), lambda i, j: (i, 0)),
        out_shape=jax.ShapeDtypeStruct((B, 1), jnp.float32),
        scratch_shapes=[
            pltpu.VMEM((A_BT, 1), jnp.float32),
            pltpu.VMEM((A_BT, 1), jnp.float32),
        ],
        compiler_params=pltpu.CompilerParams(
            dimension_semantics=("parallel", "arbitrary")),
        interpret=interpret,
    )(esum, w_bf, eb2d)
    main = pl.pallas_call(
        _out_body,
        grid=(NB, NV),
        in_specs=[
            pl.BlockSpec((BT, D), lambda i, j: (i, 0)),
            pl.BlockSpec((V_TILE, D), lambda i, j: (j, 0)),
            pl.BlockSpec((1, V_TILE), lambda i, j: (0, j)),
            pl.BlockSpec((BT, 1), lambda i, j: (i, 0)),
        ],
        out_specs=pl.BlockSpec(memory_space=pltpu.HBM),
        out_shape=jax.ShapeDtypeStruct((B, VOC), jnp.float32),
        scratch_shapes=[
            pltpu.VMEM((2, BT, V_TILE), jnp.float32),
            pltpu.SemaphoreType.DMA,
            pltpu.SemaphoreType.DMA,
        ],
        compiler_params=pltpu.CompilerParams(
            dimension_semantics=("arbitrary", "arbitrary")),
        interpret=interpret,
    )(esum, w_bf, b2d, lse)
    w_last = lax.slice(w_bf, (REM_BLK * 128, 0), (REM_BLK * 128 + 128, D))
    b_last = lax.slice(b2d, (0, REM_BLK * 128), (1, REM_BLK * 128 + 128))
    return pl.pallas_call(
        _rem_body,
        grid=(1,),
        in_specs=[
            pl.BlockSpec(memory_space=pltpu.HBM),
            pl.BlockSpec((B, D), lambda i: (0, 0)),
            pl.BlockSpec((128, D), lambda i: (0, 0)),
            pl.BlockSpec((1, 128), lambda i: (0, 0)),
            pl.BlockSpec((B, 1), lambda i: (0, 0)),
        ],
        out_specs=pl.BlockSpec((B, 128), lambda i: (0, REM_BLK)),
        out_shape=jax.ShapeDtypeStruct((B, VOC), jnp.float32),
        input_output_aliases={0: 0},
        interpret=interpret,
    )(main, esum, w_last, b_last, lse)


def kernel(x, embed_table, W, b):
    xf3 = x.astype(jnp.int32).reshape(NW, NCHUNK, CHUNK_I)
    esum = _sc_gather_sum()(xf3, embed_table).reshape(B, D)
    w_bf = jnp.pad(W.astype(jnp.bfloat16), ((0, V_PAD - VOC), (0, 0)))
    b2d = jnp.pad(b.reshape(1, VOC), ((0, 0), (0, V_PAD - VOC)))
    eb2d = jnp.pad(jnp.exp(b).reshape(1, VOC), ((0, 0), (0, V_PAD - VOC)))
    return _tc_call(esum.astype(jnp.bfloat16), w_bf, b2d, eb2d)
